# Initial kernel scaffold; baseline (speedup 1.0000x reference)
import jax
import jax.numpy as jnp
from jax.experimental import pallas as pl


def kernel(features, edge_index, W1, b1, W2, b2, Wfc, bfc):
    raise NotImplementedError("write your pallas kernel here")


# speedup vs baseline: 11.1652x; 11.1652x over previous
import jax
import jax.numpy as jnp
from jax import lax
from jax.experimental import pallas as pl
from jax.experimental.pallas import tpu as pltpu
from jax.experimental.pallas import tpu_sc as plsc

_N = 10000
_NP = 10240
_E = 320000
_NC = 2
_NS = 16
_NW = _NC * _NS
_GROUP = 128
_CHUNK = 16
_NCHUNK = 5
_EP = _NW * _NCHUNK * _CHUNK * _GROUP
_RPT = _NP // _NS

_mesh = plsc.VectorSubcoreMesh(
    core_axis_name="c", subcore_axis_name="s", num_cores=_NC, num_subcores=_NS
)
_sc_params = pltpu.CompilerParams(use_tc_tiling_on_sc=False)


def _deg_body(sidx_hbm, didx_hbm, zeros_hbm, ones_hbm, out_hbm,
              sidx_v, didx_v, ones_v, dego_sh, degi_sh, asem):
    c = lax.axis_index("c")
    s = lax.axis_index("s")
    wid = c * _NS + s
    r0 = s * _RPT
    pltpu.sync_copy(ones_hbm, ones_v)
    pltpu.sync_copy(zeros_hbm.at[pl.ds(r0, _RPT)], dego_sh.at[pl.ds(r0, _RPT)])
    pltpu.sync_copy(zeros_hbm.at[pl.ds(r0, _RPT)], degi_sh.at[pl.ds(r0, _RPT)])
    plsc.subcore_barrier()

    def chunk(k, carry):
        pltpu.sync_copy(sidx_hbm.at[wid, k], sidx_v)
        pltpu.sync_copy(didx_hbm.at[wid, k], didx_v)
        cps = []
        for j in range(_CHUNK):
            cps.append(pltpu.async_copy(ones_v, dego_sh.at[sidx_v.at[j]], asem, add=True))
            cps.append(pltpu.async_copy(ones_v, degi_sh.at[didx_v.at[j]], asem, add=True))
        for cp in cps:
            cp.wait()
        return carry

    lax.fori_loop(0, _NCHUNK, chunk, 0)
    plsc.subcore_barrier()
    pltpu.sync_copy(dego_sh.at[pl.ds(r0, _RPT)], out_hbm.at[c, 0, pl.ds(r0, _RPT)])
    pltpu.sync_copy(degi_sh.at[pl.ds(r0, _RPT)], out_hbm.at[c, 1, pl.ds(r0, _RPT)])


_deg_call = pl.kernel(
    _deg_body,
    out_type=jax.ShapeDtypeStruct((_NC, 2, _NP), jnp.float32),
    mesh=_mesh,
    scratch_types=[
        pltpu.VMEM((_CHUNK, _GROUP), jnp.int32),
        pltpu.VMEM((_CHUNK, _GROUP), jnp.int32),
        pltpu.VMEM((_GROUP,), jnp.float32),
        pltpu.VMEM_SHARED((_NP,), jnp.float32),
        pltpu.VMEM_SHARED((_NP,), jnp.float32),
        pltpu.SemaphoreType.DMA,
    ],
    compiler_params=_sc_params,
)


def _seg_body(table_hbm, sidx_hbm, didx_hbm, zeros_hbm, out_hbm,
              sidx_v, didx_v, rows_v, acc_sh, gsem, ssem):
    c = lax.axis_index("c")
    s = lax.axis_index("s")
    wid = c * _NS + s
    r0 = s * _RPT
    pltpu.sync_copy(zeros_hbm.at[pl.ds(r0, _RPT)], acc_sh.at[pl.ds(r0, _RPT)])
    plsc.subcore_barrier()

    def chunk(k, carry):
        pltpu.sync_copy(sidx_hbm.at[wid, k], sidx_v)
        pltpu.sync_copy(didx_hbm.at[wid, k], didx_v)
        gcps = [pltpu.async_copy(table_hbm.at[sidx_v.at[j]], rows_v.at[j], gsem)
                for j in range(_CHUNK)]
        for cp in gcps:
            cp.wait()
        scps = [pltpu.async_copy(rows_v.at[j], acc_sh.at[didx_v.at[j]], ssem, add=True)
                for j in range(_CHUNK)]
        for cp in scps:
            cp.wait()
        return carry

    lax.fori_loop(0, _NCHUNK, chunk, 0)
    plsc.subcore_barrier()
    pltpu.sync_copy(acc_sh.at[pl.ds(r0, _RPT)], out_hbm.at[c, pl.ds(r0, _RPT)])


_seg_call = pl.kernel(
    _seg_body,
    out_type=jax.ShapeDtypeStruct((_NC, _NP, 16), jnp.float32),
    mesh=_mesh,
    scratch_types=[
        pltpu.VMEM((_CHUNK, _GROUP), jnp.int32),
        pltpu.VMEM((_CHUNK, _GROUP), jnp.int32),
        pltpu.VMEM((_CHUNK, _GROUP, 16), jnp.float32),
        pltpu.VMEM_SHARED((_NP, 16), jnp.float32),
        pltpu.SemaphoreType.DMA,
        pltpu.SemaphoreType.DMA,
    ],
    compiler_params=_sc_params,
)


def _norms_body(degp_ref, ns_ref, nd_ref):
    dego = degp_ref[0, 0] + degp_ref[1, 0]
    degi = degp_ref[0, 1] + degp_ref[1, 1]
    ns = jnp.where(dego > 0, lax.rsqrt(jnp.maximum(dego, 1.0)), 0.0)
    nd = jnp.where(degi > 0, lax.rsqrt(jnp.maximum(degi, 1.0)), 0.0)
    ns_ref[...] = jnp.broadcast_to(ns[:, :, None], (_NP // 128, 128, 16))
    nd_ref[...] = jnp.broadcast_to(nd[:, :, None], (_NP // 128, 128, 16))


_norms_call = pl.pallas_call(
    _norms_body,
    out_shape=(
        jax.ShapeDtypeStruct((_NP // 128, 128, 16), jnp.float32),
        jax.ShapeDtypeStruct((_NP // 128, 128, 16), jnp.float32),
    ),
)


def _mm1_body(f_ref, w1_ref, ns_ref, h_ref):
    h_ref[...] = jnp.dot(f_ref[...], w1_ref[...],
                         preferred_element_type=jnp.float32) * ns_ref[...]


_mm1_call = pl.pallas_call(
    _mm1_body,
    out_shape=jax.ShapeDtypeStruct((_NP, 16), jnp.float32),
)


def _mid_body(aggp_ref, nd_ref, ns_ref, b1_ref, x_ref):
    agg = aggp_ref[0] + aggp_ref[1]
    x_ref[...] = jnp.maximum(agg * nd_ref[...] + b1_ref[...], 0.0) * ns_ref[...]


_mid_call = pl.pallas_call(
    _mid_body,
    out_shape=jax.ShapeDtypeStruct((_NP, 16), jnp.float32),
)


def _head_body(aggp_ref, nd_ref, w2_ref, wfc_ref, b2_ref, bfc_ref, o_ref):
    agg = (aggp_ref[0] + aggp_ref[1]) * nd_ref[...]
    cw = jnp.dot(w2_ref[...], wfc_ref[...], preferred_element_type=jnp.float32)
    d = jnp.dot(b2_ref[...], wfc_ref[...], preferred_element_type=jnp.float32) + bfc_ref[...]
    o_ref[...] = jnp.dot(agg[:_N], cw, preferred_element_type=jnp.float32) + d


_head_call = pl.pallas_call(
    _head_body,
    out_shape=jax.ShapeDtypeStruct((_N, 3), jnp.float32),
)


def kernel(features, edge_index, W1, b1, W2, b2, Wfc, bfc):
    src = edge_index[0].astype(jnp.int32)
    dst = edge_index[1].astype(jnp.int32)
    pad = jnp.full((_EP - _E,), _N, dtype=jnp.int32)
    sidx = jnp.concatenate([src, pad]).reshape(_NW, _NCHUNK, _CHUNK, _GROUP)
    didx = jnp.concatenate([dst, pad]).reshape(_NW, _NCHUNK, _CHUNK, _GROUP)
    z1 = jnp.zeros((_NP,), jnp.float32)
    z16 = jnp.zeros((_NP, 16), jnp.float32)
    ones = jnp.ones((_GROUP,), jnp.float32)

    degp = _deg_call(sidx, didx, z1, ones)
    ns3, nd3 = _norms_call(degp.reshape(_NC, 2, _NP // 128, 128))
    ns16 = ns3.reshape(_NP, 16)
    nd16 = nd3.reshape(_NP, 16)

    f_pad = jnp.concatenate(
        [features, jnp.zeros((_NP - _N, features.shape[1]), jnp.float32)], axis=0)
    h1s = _mm1_call(f_pad, W1, ns16)
    agg1p = _seg_call(h1s, sidx, didx, z16)
    x1s = _mid_call(agg1p, nd16, ns16, b1.reshape(1, 16))
    agg2p = _seg_call(x1s, sidx, didx, z16)
    out = _head_call(agg2p, nd16, W2, Wfc, b2.reshape(1, 128), bfc.reshape(1, 3))
    return out


# speedup vs baseline: 14.4897x; 1.2978x over previous
import jax
import jax.numpy as jnp
from jax import lax
from jax.experimental import pallas as pl
from jax.experimental.pallas import tpu as pltpu
from jax.experimental.pallas import tpu_sc as plsc

_N = 10000
_NP = 10240
_E = 320000
_NC = 2
_NS = 16
_GROUP = 128
_CHUNK = 16
_C0 = 7
_C1 = 3
_NCMAX = _C0
_TOTCHUNK = _NS * (_C0 + _C1)
_EP = _TOTCHUNK * _CHUNK * _GROUP
_DEPTH = 3
_RPT = _NP // _NS

_mesh = plsc.VectorSubcoreMesh(
    core_axis_name="c", subcore_axis_name="s", num_cores=_NC, num_subcores=_NS
)
_sc_params = pltpu.CompilerParams(use_tc_tiling_on_sc=False)


def _worker_chunks():
    c = lax.axis_index("c")
    s = lax.axis_index("s")
    nch = jnp.where(c == 0, _C0, _C1)
    base = jnp.where(c == 0, s * _C0, _NS * _C0 + s * _C1)
    return c, s, nch, base


def _deg_body(sidx_hbm, didx_hbm, zeros_hbm, ones_hbm, out_hbm,
              sidx_v, didx_v, ones_v, dego_sh, degi_sh, isem, ssem):
    c, s, nch, base = _worker_chunks()
    r0 = s * _RPT
    pltpu.sync_copy(ones_hbm, ones_v)
    pltpu.sync_copy(zeros_hbm.at[pl.ds(r0, _RPT)], dego_sh.at[pl.ds(r0, _RPT)])
    pltpu.sync_copy(zeros_hbm.at[pl.ds(r0, _RPT)], degi_sh.at[pl.ds(r0, _RPT)])
    plsc.subcore_barrier()

    idx_d = {}
    scat_d = {}
    for k in range(_NCMAX):
        d = k % _DEPTH
        idx_d[k] = (
            pltpu.make_async_copy(sidx_hbm.at[base + k], sidx_v.at[d], isem),
            pltpu.make_async_copy(didx_hbm.at[base + k], didx_v.at[d], isem),
        )
        cps = []
        for j in range(_CHUNK):
            cps.append(pltpu.make_async_copy(ones_v, dego_sh.at[sidx_v.at[d, j]], ssem))
            cps.append(pltpu.make_async_copy(ones_v, degi_sh.at[didx_v.at[d, j]], ssem))
        scat_d[k] = cps

    for cp in idx_d[0]:
        cp.start()
    for k in range(_NCMAX):
        @pl.when(k < nch)
        def _(k=k):
            for cp in idx_d[k]:
                cp.wait()
            if k >= 2:
                for cp in scat_d[k - 2]:
                    cp.wait()

        if k + 1 < _NCMAX:
            @pl.when(k + 1 < nch)
            def _(k=k):
                for cp in idx_d[k + 1]:
                    cp.start()

        @pl.when(k < nch)
        def _(k=k):
            for cp in scat_d[k]:
                cp.start(add=True)

    for k in range(_NCMAX):
        @pl.when((k < nch) & (k + 2 >= nch))
        def _(k=k):
            for cp in scat_d[k]:
                cp.wait()

    plsc.subcore_barrier()
    pltpu.sync_copy(dego_sh.at[pl.ds(r0, _RPT)], out_hbm.at[c, 0, pl.ds(r0, _RPT)])
    pltpu.sync_copy(degi_sh.at[pl.ds(r0, _RPT)], out_hbm.at[c, 1, pl.ds(r0, _RPT)])


_deg_call = pl.kernel(
    _deg_body,
    out_type=jax.ShapeDtypeStruct((_NC, 2, _NP), jnp.float32),
    mesh=_mesh,
    scratch_types=[
        pltpu.VMEM((_DEPTH, _CHUNK, _GROUP), jnp.int32),
        pltpu.VMEM((_DEPTH, _CHUNK, _GROUP), jnp.int32),
        pltpu.VMEM((_GROUP,), jnp.float32),
        pltpu.VMEM_SHARED((_NP,), jnp.float32),
        pltpu.VMEM_SHARED((_NP,), jnp.float32),
        pltpu.SemaphoreType.DMA,
        pltpu.SemaphoreType.DMA,
    ],
    compiler_params=_sc_params,
)


def _seg_body(table_hbm, sidx_hbm, didx_hbm, zeros_hbm, out_hbm,
              sidx_v, didx_v, rows_v, acc_sh, isem, gsem, ssem):
    c, s, nch, base = _worker_chunks()
    r0 = s * _RPT
    pltpu.sync_copy(zeros_hbm.at[pl.ds(r0, _RPT)], acc_sh.at[pl.ds(r0, _RPT)])
    plsc.subcore_barrier()

    idx_d = {}
    gat_d = {}
    scat_d = {}
    for k in range(_NCMAX):
        d = k % _DEPTH
        idx_d[k] = (
            pltpu.make_async_copy(sidx_hbm.at[base + k], sidx_v.at[d], isem),
            pltpu.make_async_copy(didx_hbm.at[base + k], didx_v.at[d], isem),
        )
        gat_d[k] = [pltpu.make_async_copy(table_hbm.at[sidx_v.at[d, j]],
                                          rows_v.at[k % 2, j], gsem)
                    for j in range(_CHUNK)]
        scat_d[k] = [pltpu.make_async_copy(rows_v.at[k % 2, j],
                                           acc_sh.at[didx_v.at[d, j]], ssem)
                     for j in range(_CHUNK)]

    for cp in idx_d[0]:
        cp.start()
    for k in range(_NCMAX):
        @pl.when(k < nch)
        def _(k=k):
            for cp in idx_d[k]:
                cp.wait()
            if k >= 2:
                for cp in scat_d[k - 2]:
                    cp.wait()
            for cp in gat_d[k]:
                cp.start()

        if k + 1 < _NCMAX:
            @pl.when(k + 1 < nch)
            def _(k=k):
                for cp in idx_d[k + 1]:
                    cp.start()

        @pl.when(k < nch)
        def _(k=k):
            for cp in gat_d[k]:
                cp.wait()
            for cp in scat_d[k]:
                cp.start(add=True)

    for k in range(_NCMAX):
        @pl.when((k < nch) & (k + 2 >= nch))
        def _(k=k):
            for cp in scat_d[k]:
                cp.wait()

    plsc.subcore_barrier()
    pltpu.sync_copy(acc_sh.at[pl.ds(r0, _RPT)], out_hbm.at[c, pl.ds(r0, _RPT)])


_seg_call = pl.kernel(
    _seg_body,
    out_type=jax.ShapeDtypeStruct((_NC, _NP, 16), jnp.float32),
    mesh=_mesh,
    scratch_types=[
        pltpu.VMEM((_DEPTH, _CHUNK, _GROUP), jnp.int32),
        pltpu.VMEM((_DEPTH, _CHUNK, _GROUP), jnp.int32),
        pltpu.VMEM((2, _CHUNK, _GROUP, 16), jnp.float32),
        pltpu.VMEM_SHARED((_NP, 16), jnp.float32),
        pltpu.SemaphoreType.DMA,
        pltpu.SemaphoreType.DMA,
        pltpu.SemaphoreType.DMA,
    ],
    compiler_params=_sc_params,
)


def _p_body(f_ref, w1_ref, p_ref):
    p_ref[0:_N, :] = jnp.dot(f_ref[...], w1_ref[...],
                             preferred_element_type=jnp.float32)


_p_call = pl.pallas_call(
    _p_body,
    out_shape=jax.ShapeDtypeStruct((_NP, 16), jnp.float32),
)


def _norms_body(degp_ref, p3_ref, h3_ref, ns_ref, nd_ref):
    dego = degp_ref[0, 0] + degp_ref[1, 0]
    degi = degp_ref[0, 1] + degp_ref[1, 1]
    ns = jnp.where(dego > 0, lax.rsqrt(jnp.maximum(dego, 1.0)), 0.0)
    nd = jnp.where(degi > 0, lax.rsqrt(jnp.maximum(degi, 1.0)), 0.0)
    ns3 = jnp.broadcast_to(ns[:, :, None], (_NP // 128, 128, 16))
    h3_ref[...] = p3_ref[...] * ns3
    ns_ref[...] = ns3
    nd_ref[...] = jnp.broadcast_to(nd[:, :, None], (_NP // 128, 128, 16))


_norms_call = pl.pallas_call(
    _norms_body,
    out_shape=(
        jax.ShapeDtypeStruct((_NP // 128, 128, 16), jnp.float32),
        jax.ShapeDtypeStruct((_NP // 128, 128, 16), jnp.float32),
        jax.ShapeDtypeStruct((_NP // 128, 128, 16), jnp.float32),
    ),
)


def _mid_body(aggp_ref, nd_ref, ns_ref, b1_ref, x_ref):
    agg = aggp_ref[0] + aggp_ref[1]
    x_ref[...] = jnp.maximum(agg * nd_ref[...] + b1_ref[...], 0.0) * ns_ref[...]


_mid_call = pl.pallas_call(
    _mid_body,
    out_shape=jax.ShapeDtypeStruct((_NP, 16), jnp.float32),
)


def _head_body(aggp_ref, nd_ref, w2_ref, wfc_ref, b2_ref, bfc_ref, o_ref):
    agg = (aggp_ref[0] + aggp_ref[1]) * nd_ref[...]
    cw = jnp.dot(w2_ref[...], wfc_ref[...], preferred_element_type=jnp.float32)
    d = jnp.dot(b2_ref[...], wfc_ref[...], preferred_element_type=jnp.float32) + bfc_ref[...]
    o_ref[...] = jnp.dot(agg[:_N], cw, preferred_element_type=jnp.float32) + d


_head_call = pl.pallas_call(
    _head_body,
    out_shape=jax.ShapeDtypeStruct((_N, 3), jnp.float32),
)


def kernel(features, edge_index, W1, b1, W2, b2, Wfc, bfc):
    src = edge_index[0].astype(jnp.int32)
    dst = edge_index[1].astype(jnp.int32)
    pad = jnp.full((_EP - _E,), _N, dtype=jnp.int32)
    sidx = jnp.concatenate([src, pad]).reshape(_TOTCHUNK, _CHUNK, _GROUP)
    didx = jnp.concatenate([dst, pad]).reshape(_TOTCHUNK, _CHUNK, _GROUP)
    z1 = jnp.zeros((_NP,), jnp.float32)
    z16 = jnp.zeros((_NP, 16), jnp.float32)
    ones = jnp.ones((_GROUP,), jnp.float32)

    p = _p_call(features, W1)
    degp = _deg_call(sidx, didx, z1, ones)
    h3, ns3, nd3 = _norms_call(
        degp.reshape(_NC, 2, _NP // 128, 128),
        p.reshape(_NP // 128, 128, 16))
    h1s = h3.reshape(_NP, 16)
    ns16 = ns3.reshape(_NP, 16)
    nd16 = nd3.reshape(_NP, 16)

    agg1p = _seg_call(h1s, sidx, didx, z16)
    x1s = _mid_call(agg1p, nd16, ns16, b1.reshape(1, 16))
    agg2p = _seg_call(x1s, sidx, didx, z16)
    out = _head_call(agg2p, nd16, W2, Wfc, b2.reshape(1, 128), bfc.reshape(1, 3))
    return out


# speedup vs baseline: 22.2983x; 1.5389x over previous
import jax
import jax.numpy as jnp
from jax import lax
from jax.experimental import pallas as pl
from jax.experimental.pallas import tpu as pltpu
from jax.experimental.pallas import tpu_sc as plsc

_N = 10000
_NP = 10240
_E = 320000
_NC = 2
_NS = 16
_GROUP = 128
_CHUNK = 16
_C0 = 5
_C1 = 5
_NCMAX = _C0
_TOTCHUNK = _NS * (_C0 + _C1)
_EP = _TOTCHUNK * _CHUNK * _GROUP
_DEPTH = 3
_RPT = _NP // _NS

_mesh = plsc.VectorSubcoreMesh(
    core_axis_name="c", subcore_axis_name="s", num_cores=_NC, num_subcores=_NS
)
_sc_params = pltpu.CompilerParams(use_tc_tiling_on_sc=False)


def _worker_chunks():
    c = lax.axis_index("c")
    s = lax.axis_index("s")
    nch = jnp.where(c == 0, _C0, _C1)
    base = jnp.where(c == 0, s * _C0, _NS * _C0 + s * _C1)
    return c, s, nch, base


def _deg_body(sidx_hbm, didx_hbm, zeros_hbm, ones_hbm, out_hbm,
              sidx_v, didx_v, ones_v, dego_sh, degi_sh, isem, ssem):
    c, s, nch, base = _worker_chunks()
    r0 = s * _RPT
    pltpu.sync_copy(ones_hbm, ones_v)
    pltpu.sync_copy(zeros_hbm.at[pl.ds(r0, _RPT)], dego_sh.at[pl.ds(r0, _RPT)])
    pltpu.sync_copy(zeros_hbm.at[pl.ds(r0, _RPT)], degi_sh.at[pl.ds(r0, _RPT)])
    plsc.subcore_barrier()

    idx_d = {}
    scat_d = {}
    for k in range(_NCMAX):
        d = k % _DEPTH
        idx_d[k] = (
            pltpu.make_async_copy(sidx_hbm.at[base + k], sidx_v.at[d], isem),
            pltpu.make_async_copy(didx_hbm.at[base + k], didx_v.at[d], isem),
        )
        cps = []
        for j in range(_CHUNK):
            cps.append(pltpu.make_async_copy(ones_v, dego_sh.at[sidx_v.at[d, j]], ssem))
            cps.append(pltpu.make_async_copy(ones_v, degi_sh.at[didx_v.at[d, j]], ssem))
        scat_d[k] = cps

    for cp in idx_d[0]:
        cp.start()
    for k in range(_NCMAX):
        @pl.when(k < nch)
        def _(k=k):
            for cp in idx_d[k]:
                cp.wait()
            if k >= 2:
                for cp in scat_d[k - 2]:
                    cp.wait()

        if k + 1 < _NCMAX:
            @pl.when(k + 1 < nch)
            def _(k=k):
                for cp in idx_d[k + 1]:
                    cp.start()

        @pl.when(k < nch)
        def _(k=k):
            for cp in scat_d[k]:
                cp.start(add=True)

    for k in range(_NCMAX):
        @pl.when((k < nch) & (k + 2 >= nch))
        def _(k=k):
            for cp in scat_d[k]:
                cp.wait()

    plsc.subcore_barrier()
    pltpu.sync_copy(dego_sh.at[pl.ds(r0, _RPT)], out_hbm.at[c, 0, pl.ds(r0, _RPT)])
    pltpu.sync_copy(degi_sh.at[pl.ds(r0, _RPT)], out_hbm.at[c, 1, pl.ds(r0, _RPT)])


_deg_call = pl.kernel(
    _deg_body,
    out_type=jax.ShapeDtypeStruct((_NC, 2, _NP), jnp.float32),
    mesh=_mesh,
    scratch_types=[
        pltpu.VMEM((_DEPTH, _CHUNK, _GROUP), jnp.int32),
        pltpu.VMEM((_DEPTH, _CHUNK, _GROUP), jnp.int32),
        pltpu.VMEM((_GROUP,), jnp.float32),
        pltpu.VMEM_SHARED((_NP,), jnp.float32),
        pltpu.VMEM_SHARED((_NP,), jnp.float32),
        pltpu.SemaphoreType.DMA,
        pltpu.SemaphoreType.DMA,
    ],
    compiler_params=_sc_params,
)


def _seg_body(table_hbm, sidx_hbm, didx_hbm, zeros_hbm, out_hbm,
              sidx_v, didx_v, rows_v, acc_sh, isem, gsem, ssem):
    c, s, nch, base = _worker_chunks()
    r0 = s * _RPT
    pltpu.sync_copy(zeros_hbm.at[pl.ds(r0, _RPT)], acc_sh.at[pl.ds(r0, _RPT)])
    plsc.subcore_barrier()

    idx_d = {}
    gat_d = {}
    scat_d = {}
    for k in range(_NCMAX):
        d = k % _DEPTH
        idx_d[k] = (
            pltpu.make_async_copy(sidx_hbm.at[base + k], sidx_v.at[d], isem),
            pltpu.make_async_copy(didx_hbm.at[base + k], didx_v.at[d], isem),
        )
        gat_d[k] = [pltpu.make_async_copy(table_hbm.at[sidx_v.at[d, j]],
                                          rows_v.at[k % 2, j], gsem)
                    for j in range(_CHUNK)]
        scat_d[k] = [pltpu.make_async_copy(rows_v.at[k % 2, j],
                                           acc_sh.at[didx_v.at[d, j]], ssem)
                     for j in range(_CHUNK)]

    for cp in idx_d[0]:
        cp.start()
    for k in range(_NCMAX):
        @pl.when(k < nch)
        def _(k=k):
            for cp in idx_d[k]:
                cp.wait()
            if k >= 2:
                for cp in scat_d[k - 2]:
                    cp.wait()
            for cp in gat_d[k]:
                cp.start()

        if k + 1 < _NCMAX:
            @pl.when(k + 1 < nch)
            def _(k=k):
                for cp in idx_d[k + 1]:
                    cp.start()

        @pl.when(k < nch)
        def _(k=k):
            for cp in gat_d[k]:
                cp.wait()
            for cp in scat_d[k]:
                cp.start(add=True)

    for k in range(_NCMAX):
        @pl.when((k < nch) & (k + 2 >= nch))
        def _(k=k):
            for cp in scat_d[k]:
                cp.wait()

    plsc.subcore_barrier()
    pltpu.sync_copy(acc_sh.at[pl.ds(r0, _RPT)], out_hbm.at[c, pl.ds(r0, _RPT)])


_seg_call = pl.kernel(
    _seg_body,
    out_type=jax.ShapeDtypeStruct((_NC, _NP, 16), jnp.float32),
    mesh=_mesh,
    scratch_types=[
        pltpu.VMEM((_DEPTH, _CHUNK, _GROUP), jnp.int32),
        pltpu.VMEM((_DEPTH, _CHUNK, _GROUP), jnp.int32),
        pltpu.VMEM((2, _CHUNK, _GROUP, 16), jnp.float32),
        pltpu.VMEM_SHARED((_NP, 16), jnp.float32),
        pltpu.SemaphoreType.DMA,
        pltpu.SemaphoreType.DMA,
        pltpu.SemaphoreType.DMA,
    ],
    compiler_params=_sc_params,
)


def _p_body(f_ref, w1_ref, p_ref):
    p_ref[0:_N, :] = jnp.dot(f_ref[...], w1_ref[...],
                             preferred_element_type=jnp.float32)


_p_call = pl.pallas_call(
    _p_body,
    out_shape=jax.ShapeDtypeStruct((_NP, 16), jnp.float32),
)


def _norms_body(degp_ref, p3_ref, h3_ref, ns_ref, nd_ref):
    dego = degp_ref[0, 0] + degp_ref[1, 0]
    degi = degp_ref[0, 1] + degp_ref[1, 1]
    ns = jnp.where(dego > 0, lax.rsqrt(jnp.maximum(dego, 1.0)), 0.0)
    nd = jnp.where(degi > 0, lax.rsqrt(jnp.maximum(degi, 1.0)), 0.0)
    ns3 = jnp.broadcast_to(ns[:, :, None], (_NP // 128, 128, 16))
    h3_ref[...] = p3_ref[...] * ns3
    ns_ref[...] = ns3
    nd_ref[...] = jnp.broadcast_to(nd[:, :, None], (_NP // 128, 128, 16))


_norms_call = pl.pallas_call(
    _norms_body,
    out_shape=(
        jax.ShapeDtypeStruct((_NP // 128, 128, 16), jnp.float32),
        jax.ShapeDtypeStruct((_NP // 128, 128, 16), jnp.float32),
        jax.ShapeDtypeStruct((_NP // 128, 128, 16), jnp.float32),
    ),
)


def _mid_body(aggp_ref, nd_ref, ns_ref, b1_ref, x_ref):
    agg = aggp_ref[0] + aggp_ref[1]
    x_ref[...] = jnp.maximum(agg * nd_ref[...] + b1_ref[...], 0.0) * ns_ref[...]


_mid_call = pl.pallas_call(
    _mid_body,
    out_shape=jax.ShapeDtypeStruct((_NP, 16), jnp.float32),
)


def _head_body(aggp_ref, nd_ref, w2_ref, wfc_ref, b2_ref, bfc_ref, o_ref):
    agg = (aggp_ref[0] + aggp_ref[1]) * nd_ref[...]
    cw = jnp.dot(w2_ref[...], wfc_ref[...], preferred_element_type=jnp.float32)
    d = jnp.dot(b2_ref[...], wfc_ref[...], preferred_element_type=jnp.float32) + bfc_ref[...]
    o_ref[...] = jnp.dot(agg[:_N], cw, preferred_element_type=jnp.float32) + d


_head_call = pl.pallas_call(
    _head_body,
    out_shape=jax.ShapeDtypeStruct((_N, 3), jnp.float32),
)


def kernel(features, edge_index, W1, b1, W2, b2, Wfc, bfc):
    pad = _N + (jnp.arange(_EP - _E, dtype=jnp.int32) % (_NP - _N))
    ei = jnp.concatenate(
        [edge_index.astype(jnp.int32), jnp.stack([pad, pad])], axis=1)
    sidx = ei[0].reshape(_TOTCHUNK, _CHUNK, _GROUP)
    didx = ei[1].reshape(_TOTCHUNK, _CHUNK, _GROUP)
    z1 = jnp.zeros((_NP,), jnp.float32)
    z16 = jnp.zeros((_NP, 16), jnp.float32)
    ones = jnp.ones((_GROUP,), jnp.float32)

    p = _p_call(features, W1)
    degp = _deg_call(sidx, didx, z1, ones)
    h3, ns3, nd3 = _norms_call(
        degp.reshape(_NC, 2, _NP // 128, 128),
        p.reshape(_NP // 128, 128, 16))
    h1s = h3.reshape(_NP, 16)
    ns16 = ns3.reshape(_NP, 16)
    nd16 = nd3.reshape(_NP, 16)

    agg1p = _seg_call(h1s, sidx, didx, z16)
    x1s = _mid_call(agg1p, nd16, ns16, b1.reshape(1, 16))
    agg2p = _seg_call(x1s, sidx, didx, z16)
    out = _head_call(agg2p, nd16, W2, Wfc, b2.reshape(1, 128), bfc.reshape(1, 3))
    return out
